# baseline (speedup 1.0000x reference)
import functools

import jax
import jax.numpy as jnp
from jax import lax
from jax.experimental import pallas as pl
from jax.experimental.pallas import tpu as pltpu
from jax.experimental.pallas import tpu_sc as plsc

N_NODES = 10000
N_EDGES = 320000
D = 128

NC = 2
NS = 16
NW = NC * NS
E_PER_W = N_EDGES // NW
BATCH = 80
N_BATCH = E_PER_W // BATCH
ROWS_PER_TILE = 624
TAIL_ROWS = N_NODES - NS * ROWS_PER_TILE


def _sc_aggregate(xi, ei, attr):
    mesh = plsc.VectorSubcoreMesh(core_axis_name="c", subcore_axis_name="s")

    @functools.partial(
        pl.kernel,
        mesh=mesh,
        compiler_params=pltpu.CompilerParams(use_tc_tiling_on_sc=False),
        out_type=jax.ShapeDtypeStruct((NC, N_NODES, D), jnp.float32),
        scratch_types=[
            pltpu.VMEM((4, BATCH), jnp.int32),
            pltpu.VMEM((4, BATCH), jnp.int32),
            pltpu.VMEM((4, BATCH), jnp.float32),
            pltpu.VMEM((BATCH, D // 2), jnp.int32),
            pltpu.VMEM((BATCH, D // 2), jnp.int32),
            pltpu.VMEM((BATCH, D), jnp.float32),
            pltpu.VMEM((BATCH, D), jnp.float32),
            pltpu.VMEM_SHARED((N_NODES, D), jnp.float32),
            pltpu.SemaphoreType.DMA,
            pltpu.SemaphoreType.DMA,
            pltpu.SemaphoreType.DMA,
            pltpu.SemaphoreType.DMA,
            pltpu.SemaphoreType.DMA,
            pltpu.SemaphoreType.DMA,
        ],
    )
    def sc_kernel(xi_hbm, ei_hbm, attr_hbm, out_hbm,
                  sidx, didx, attrs, gb0, gb1, sb0, sb1, agg_sh,
                  gsem0, gsem1, ssem0, ssem1, isem0, isem1):
        cid = lax.axis_index("c")
        sid = lax.axis_index("s")
        wid = cid * NS + sid
        gb = (gb0, gb1)
        sb = (sb0, sb1)
        gsem = (gsem0, gsem1)
        ssem = (ssem0, ssem1)
        isem = (isem0, isem1)

        @plsc.parallel_loop(0, BATCH)
        def _(j):
            for c in range(D // 16):
                sb0[j, pl.ds(c * 16, 16)] = jnp.zeros((16,), jnp.float32)
        r0 = pl.multiple_of(sid * ROWS_PER_TILE, 8)
        for z in range(ROWS_PER_TILE // BATCH):
            pltpu.sync_copy(sb0, agg_sh.at[pl.ds(r0 + z * BATCH, BATCH)])
        zrem = ROWS_PER_TILE - (ROWS_PER_TILE // BATCH) * BATCH
        pltpu.sync_copy(sb0.at[pl.ds(0, zrem)],
                        agg_sh.at[pl.ds(r0 + ROWS_PER_TILE - zrem, zrem)])
        t0 = NS * ROWS_PER_TILE
        @pl.when(sid == NS - 1)
        def _():
            pltpu.sync_copy(sb0.at[pl.ds(0, TAIL_ROWS)],
                            agg_sh.at[pl.ds(t0, TAIL_ROWS)])
        plsc.subcore_barrier()

        def ebase(ii):
            return pl.multiple_of((wid * N_BATCH + ii) * BATCH, 8)

        def idx_start(ii, slot, b):
            base = ebase(ii)
            pltpu.async_copy(ei_hbm.at[pl.ds(base, BATCH)], sidx.at[slot],
                             isem[b])
            pltpu.async_copy(ei_hbm.at[pl.ds(N_EDGES + base, BATCH)],
                             didx.at[slot], isem[b])
            pltpu.async_copy(attr_hbm.at[pl.ds(base, BATCH)], attrs.at[slot],
                             isem[b])

        def idx_wait(b):
            pltpu.make_async_copy(ei_hbm.at[pl.ds(0, BATCH)], sidx.at[0],
                                  isem[b]).wait()
            pltpu.make_async_copy(ei_hbm.at[pl.ds(0, BATCH)], didx.at[0],
                                  isem[b]).wait()
            pltpu.make_async_copy(attr_hbm.at[pl.ds(0, BATCH)], attrs.at[0],
                                  isem[b]).wait()

        def gather_start(ii, slot, b):
            pltpu.async_copy(xi_hbm.at[sidx.at[slot]], gb[b], gsem[b])

        def gather_wait(b):
            pltpu.make_async_copy(xi_hbm.at[sidx.at[0]], gb[b],
                                  gsem[b]).wait()

        def scatter_start(ii, slot, b):
            pltpu.async_copy(sb[b], agg_sh.at[didx.at[slot]], ssem[b],
                             add=True)

        def scatter_wait(b):
            pltpu.make_async_copy(sb[b], agg_sh.at[didx.at[0]],
                                  ssem[b]).wait()

        def scale(slot, b):
            gbuf, sbuf = gb[b], sb[b]
            hi_mask = jnp.full((16,), -65536, jnp.int32)
            @plsc.parallel_loop(0, BATCH // 16)
            def _(g):
                av = attrs[slot, pl.ds(g * 16, 16)]
                for l in range(16):
                    a = av[l]
                    j = g * 16 + l
                    for c in range(D // 32):
                        v = gbuf[j, pl.ds(c * 16, 16)]
                        even = lax.bitcast_convert_type(
                            lax.shift_left(v, 16), jnp.float32)
                        odd = lax.bitcast_convert_type(
                            lax.bitwise_and(v, hi_mask), jnp.float32)
                        sbuf[j, pl.ds(c * 32, 16)] = even * a
                        sbuf[j, pl.ds(c * 32 + 16, 16)] = odd * a

        for p in range(2):
            bp = ebase(p)
            pltpu.sync_copy(ei_hbm.at[pl.ds(bp, BATCH)], sidx.at[p])
            pltpu.sync_copy(ei_hbm.at[pl.ds(N_EDGES + bp, BATCH)],
                            didx.at[p])
            pltpu.sync_copy(attr_hbm.at[pl.ds(bp, BATCH)], attrs.at[p])
        gather_start(0, 0, 0)
        gather_start(1, 1, 1)

        def substep(i, q):
            ii = 4 * i + q
            slot, b = q, q % 2
            nslot = (q + 2) % 4
            gather_wait(b)
            if q >= 2:
                scatter_wait(b)
            else:
                @pl.when(i > 0)
                def _():
                    scatter_wait(b)
            if q == 3:
                @pl.when(i < N_BATCH // 4 - 1)
                def _():
                    idx_start(ii + 2, nslot, b)
            else:
                idx_start(ii + 2, nslot, b)
            scale(slot, b)
            scatter_start(ii, slot, b)
            if q == 3:
                @pl.when(i < N_BATCH // 4 - 1)
                def _():
                    idx_wait(b)
                    gather_start(ii + 2, nslot, b)
            else:
                idx_wait(b)
                gather_start(ii + 2, nslot, b)

        def step(i):
            for q in range(4):
                substep(i, q)
        lax.fori_loop(0, N_BATCH // 4, lambda i, _: (step(i), 0)[1], 0)

        gather_wait(0)
        scatter_wait(0)
        scale(0, 0)
        scatter_start(N_BATCH - 1, 0, 0)
        scatter_wait(0)
        scatter_wait(1)

        plsc.subcore_barrier()
        pltpu.sync_copy(agg_sh.at[pl.ds(r0, ROWS_PER_TILE)],
                        out_hbm.at[cid, pl.ds(r0, ROWS_PER_TILE)])
        @pl.when(sid == NS - 1)
        def _():
            pltpu.sync_copy(agg_sh.at[pl.ds(t0, TAIL_ROWS)],
                            out_hbm.at[cid, pl.ds(t0, TAIL_ROWS)])

    return sc_kernel(xi, ei, attr)


def _tc_body(p_ref, w_ref, b_ref, g_ref, be_ref, o_ref):
    agg = p_ref[0] + p_ref[1]
    h = jnp.dot(agg, w_ref[:], preferred_element_type=jnp.float32) + b_ref[:]
    mean = jnp.mean(h, axis=0, keepdims=True)
    var = jnp.mean((h - mean) ** 2, axis=0, keepdims=True)
    hn = (h - mean) * lax.rsqrt(var + 1e-5) * g_ref[:] + be_ref[:]
    norm = jnp.sqrt(jnp.sum(hn * hn, axis=-1, keepdims=True))
    o_ref[:] = hn / norm


def kernel(x, edge_index, edge_attr, W, b, gamma, beta):
    ei = edge_index.astype(jnp.int32).reshape(2 * N_EDGES)
    attr = edge_attr.astype(jnp.float32).reshape(N_EDGES)
    xi = lax.bitcast_convert_type(
        x.astype(jnp.bfloat16).reshape(N_NODES, D // 2, 2), jnp.int32)
    W_perm = W.reshape(4, 16, 2, D).transpose(0, 2, 1, 3).reshape(D, D)

    parts = _sc_aggregate(xi, ei, attr)

    out = pl.pallas_call(
        _tc_body,
        out_shape=jax.ShapeDtypeStruct((N_NODES, D), jnp.float32),
    )(parts, W_perm, b.reshape(1, D), gamma.reshape(1, D), beta.reshape(1, D))
    return out


# speedup vs baseline: 1.6244x; 1.6244x over previous
import functools

import jax
import jax.numpy as jnp
from jax import lax
from jax.experimental import pallas as pl
from jax.experimental.pallas import tpu as pltpu
from jax.experimental.pallas import tpu_sc as plsc

N_NODES = 10000
N_EDGES = 320000
D = 128

NC = 2
NS = 16
NW = NC * NS
E_PER_W = N_EDGES // NW
BATCH = 80
N_BATCH = E_PER_W // BATCH
ROWS_PER_TILE = 624
TAIL_ROWS = N_NODES - NS * ROWS_PER_TILE


def _sc_aggregate(x, ei, attr):
    mesh = plsc.VectorSubcoreMesh(core_axis_name="c", subcore_axis_name="s")

    @functools.partial(
        pl.kernel,
        mesh=mesh,
        out_type=jax.ShapeDtypeStruct((NC, N_NODES, D), jnp.float32),
        scratch_types=[
            pltpu.VMEM((4, BATCH), jnp.int32),
            pltpu.VMEM((4, BATCH), jnp.int32),
            pltpu.VMEM((4, BATCH), jnp.float32),
            pltpu.VMEM((BATCH, D), jnp.float32),
            pltpu.VMEM((BATCH, D), jnp.float32),
            pltpu.VMEM((BATCH, D), jnp.float32),
            pltpu.VMEM((BATCH, D), jnp.float32),
            pltpu.VMEM_SHARED((N_NODES, D), jnp.float32),
            pltpu.SemaphoreType.DMA,
            pltpu.SemaphoreType.DMA,
            pltpu.SemaphoreType.DMA,
            pltpu.SemaphoreType.DMA,
            pltpu.SemaphoreType.DMA,
            pltpu.SemaphoreType.DMA,
        ],
    )
    def sc_kernel(x_hbm, ei_hbm, attr_hbm, out_hbm,
                  sidx, didx, attrs, gb0, gb1, sb0, sb1, agg_sh,
                  gsem0, gsem1, ssem0, ssem1, isem0, isem1):
        cid = lax.axis_index("c")
        sid = lax.axis_index("s")
        wid = cid * NS + sid
        gb = (gb0, gb1)
        sb = (sb0, sb1)
        gsem = (gsem0, gsem1)
        ssem = (ssem0, ssem1)
        isem = (isem0, isem1)

        def ebase(ii):
            return pl.multiple_of((wid * N_BATCH + ii) * BATCH, 8)

        def gather_start(ii, slot, b):
            pltpu.async_copy(x_hbm.at[sidx.at[slot]], gb[b], gsem[b])

        for p in range(2):
            bp = ebase(p)
            pltpu.sync_copy(ei_hbm.at[pl.ds(bp, BATCH)], sidx.at[p])
            pltpu.sync_copy(ei_hbm.at[pl.ds(N_EDGES + bp, BATCH)],
                            didx.at[p])
            pltpu.sync_copy(attr_hbm.at[pl.ds(bp, BATCH)], attrs.at[p])
        gather_start(0, 0, 0)
        gather_start(1, 1, 1)

        @plsc.parallel_loop(0, BATCH)
        def _(j):
            for c in range(D // 16):
                sb0[j, pl.ds(c * 16, 16)] = jnp.zeros((16,), jnp.float32)
        r0 = pl.multiple_of(sid * ROWS_PER_TILE, 8)
        for z in range(ROWS_PER_TILE // BATCH):
            pltpu.sync_copy(sb0, agg_sh.at[pl.ds(r0 + z * BATCH, BATCH)])
        zrem = ROWS_PER_TILE - (ROWS_PER_TILE // BATCH) * BATCH
        pltpu.sync_copy(sb0.at[pl.ds(0, zrem)],
                        agg_sh.at[pl.ds(r0 + ROWS_PER_TILE - zrem, zrem)])
        t0 = NS * ROWS_PER_TILE
        @pl.when(sid == NS - 1)
        def _():
            pltpu.sync_copy(sb0.at[pl.ds(0, TAIL_ROWS)],
                            agg_sh.at[pl.ds(t0, TAIL_ROWS)])
        plsc.subcore_barrier()

        def idx_start(ii, slot, b):
            base = ebase(ii)
            pltpu.async_copy(ei_hbm.at[pl.ds(base, BATCH)], sidx.at[slot],
                             isem[b])
            pltpu.async_copy(ei_hbm.at[pl.ds(N_EDGES + base, BATCH)],
                             didx.at[slot], isem[b])
            pltpu.async_copy(attr_hbm.at[pl.ds(base, BATCH)], attrs.at[slot],
                             isem[b])

        def idx_wait(b):
            pltpu.make_async_copy(ei_hbm.at[pl.ds(0, BATCH)], sidx.at[0],
                                  isem[b]).wait()
            pltpu.make_async_copy(ei_hbm.at[pl.ds(0, BATCH)], didx.at[0],
                                  isem[b]).wait()
            pltpu.make_async_copy(attr_hbm.at[pl.ds(0, BATCH)], attrs.at[0],
                                  isem[b]).wait()

        def gather_wait(b):
            pltpu.make_async_copy(x_hbm.at[sidx.at[0]], gb[b],
                                  gsem[b]).wait()

        def scatter_start(ii, slot, b):
            pltpu.async_copy(sb[b], agg_sh.at[didx.at[slot]], ssem[b],
                             add=True)

        def scatter_wait(b):
            pltpu.make_async_copy(sb[b], agg_sh.at[didx.at[0]],
                                  ssem[b]).wait()

        def scale(slot, b):
            gbuf, sbuf = gb[b], sb[b]
            @plsc.parallel_loop(0, BATCH // 16)
            def _(g):
                av = attrs[slot, pl.ds(g * 16, 16)]
                for l in range(16):
                    a = av[l]
                    j = g * 16 + l
                    for c in range(D // 16):
                        sl = pl.ds(c * 16, 16)
                        sbuf[j, sl] = gbuf[j, sl] * a

        def substep(i, q):
            ii = 4 * i + q
            slot, b = q, q % 2
            nslot = (q + 2) % 4
            gather_wait(b)
            if q >= 2:
                scatter_wait(b)
            else:
                @pl.when(i > 0)
                def _():
                    scatter_wait(b)
            if q == 3:
                @pl.when(i < N_BATCH // 4 - 1)
                def _():
                    idx_start(ii + 2, nslot, b)
            else:
                idx_start(ii + 2, nslot, b)
            scale(slot, b)
            scatter_start(ii, slot, b)
            if q == 3:
                @pl.when(i < N_BATCH // 4 - 1)
                def _():
                    idx_wait(b)
                    gather_start(ii + 2, nslot, b)
            else:
                idx_wait(b)
                gather_start(ii + 2, nslot, b)

        def step(i):
            for q in range(4):
                substep(i, q)
        lax.fori_loop(0, N_BATCH // 4, lambda i, _: (step(i), 0)[1], 0)

        gather_wait(0)
        scatter_wait(0)
        scale(0, 0)
        scatter_start(N_BATCH - 1, 0, 0)
        scatter_wait(0)
        scatter_wait(1)

        plsc.subcore_barrier()
        pltpu.sync_copy(agg_sh.at[pl.ds(r0, ROWS_PER_TILE)],
                        out_hbm.at[cid, pl.ds(r0, ROWS_PER_TILE)])
        @pl.when(sid == NS - 1)
        def _():
            pltpu.sync_copy(agg_sh.at[pl.ds(t0, TAIL_ROWS)],
                            out_hbm.at[cid, pl.ds(t0, TAIL_ROWS)])

    return sc_kernel(x, ei, attr)


def _tc_body(p_ref, w_ref, b_ref, g_ref, be_ref, o_ref):
    agg = p_ref[0] + p_ref[1]
    h = jnp.dot(agg, w_ref[:], preferred_element_type=jnp.float32) + b_ref[:]
    mean = jnp.mean(h, axis=0, keepdims=True)
    var = jnp.mean((h - mean) ** 2, axis=0, keepdims=True)
    hn = (h - mean) * lax.rsqrt(var + 1e-5) * g_ref[:] + be_ref[:]
    norm = jnp.sqrt(jnp.sum(hn * hn, axis=-1, keepdims=True))
    o_ref[:] = hn / norm


def kernel(x, edge_index, edge_attr, W, b, gamma, beta):
    ei = edge_index.astype(jnp.int32).reshape(2 * N_EDGES)
    attr = edge_attr.astype(jnp.float32).reshape(N_EDGES)

    parts = _sc_aggregate(x, ei, attr)

    out = pl.pallas_call(
        _tc_body,
        out_shape=jax.ShapeDtypeStruct((N_NODES, D), jnp.float32),
    )(parts, W, b.reshape(1, D), gamma.reshape(1, D), beta.reshape(1, D))
    return out
